# baseline (speedup 1.0000x reference)
import jax
import jax.numpy as jnp
from jax.experimental import pallas as pl
from jax.experimental.pallas import tpu as pltpu

N = 2048
D = 384
CD = 2 * 49 * 3 * 3
R = 256
NB = N // R
F32 = jnp.float32
BF16 = jnp.bfloat16
PREC = jax.lax.Precision.DEFAULT


def _dot(a, b):
    return jax.lax.dot_general(a, b, (((1,), (0,)), ((), ())),
                               precision=PREC, preferred_element_type=F32)


def _ln(x, w, b, eps=1e-3):
    m = jnp.mean(x, axis=-1, keepdims=True)
    v = jnp.mean((x - m) ** 2, axis=-1, keepdims=True)
    return (x - m) / jnp.sqrt(v + eps) * w[None, :] + b[None, :]


def _relu(x):
    return jnp.maximum(x, 0.0)


def _split(x):
    xs = jnp.where(jnp.isnan(x), 0.0, jnp.clip(x, -3e38, 3e38))
    hi = xs.astype(BF16)
    lo = (xs - hi.astype(F32)).astype(BF16)
    return hi, lo


def _gather(idx, hf, lf):
    colid = jax.lax.broadcasted_iota(jnp.int32, (R, N), 1)
    m = (idx[:, None] == colid).astype(BF16)
    return _dot(m, hf) + _dot(m, lf)


def _segsum(blk, q, hf, lf):
    rowid = blk * R + jax.lax.broadcasted_iota(jnp.int32, (R, N), 0)
    m = (q[None, :] == rowid).astype(BF16)
    return _dot(m, hf) + _dot(m, lf)


def _mega(corr_ref, netin_ref, inpin_ref, kkf_ref, jjf_ref, iif_ref,
          kkb_ref, jjb_ref, w0_ref, wstk_ref, hw_ref, bias_ref,
          net5_ref, head_ref,
          sA, sB, sC, sD, sE, h0, l0, h1, l1, ix_ref, jx_ref):
    s = pl.program_id(0)
    stage = s // NB
    blk = s % NB
    start = pl.multiple_of(blk * R, R)
    rows = pl.ds(start, R)

    def W(k):
        return wstk_ref[k]

    def bias(k):
        return bias_ref[k, :]

    @pl.when(stage == 0)
    def _s0():
        c = _relu(_dot(corr_ref[...], w0_ref[...]) + bias(0)[None, :])
        c = _dot(c, W(0)) + bias(1)[None, :]
        c = _relu(_ln(c, bias(2), bias(3)))
        c = _dot(c, W(1)) + bias(4)[None, :]
        x = netin_ref[...] + inpin_ref[...] + c
        net1 = _ln(x, bias(5), bias(6))
        sA[rows, :] = net1
        hi, lo = _split(net1)
        h0[rows, :] = hi
        l0[rows, :] = lo
        kkf = kkf_ref[0, :]
        jjf = jjf_ref[0, :]
        kkb = kkb_ref[0, 0, :]
        jjb = jjb_ref[0, 0, :]
        mask = kkf[None, :] == kkb[:, None]
        colid = jax.lax.broadcasted_iota(jnp.int32, (R, N), 1)
        jjfc = jnp.broadcast_to(jjf[None, :], (R, N))
        jjbc = jjb[:, None]
        prev_vals = jnp.where(mask & (jjfc < jjbc), jjfc, 0)
        pm = jnp.max(prev_vals, axis=1, keepdims=True)
        ixv = jnp.min(jnp.where(prev_vals == pm, colid, N), axis=1)
        next_vals = jnp.where(mask & (jjfc > jjbc), jjfc, N)
        nm = jnp.min(next_vals, axis=1, keepdims=True)
        jxv = jnp.min(jnp.where(next_vals == nm, colid, N), axis=1)
        ix_ref[blk, :] = jnp.clip(ixv, 0, N - 1)
        jx_ref[blk, :] = jnp.clip(jxv, 0, N - 1)

    @pl.when(stage == 1)
    def _s1():
        g = _gather(ix_ref[blk, :], h0[...], l0[...])
        h = _relu(_dot(g, W(2)) + bias(7)[None, :])
        net2 = sA[rows, :] + _dot(h, W(3)) + bias(8)[None, :]
        sB[rows, :] = net2
        hi, lo = _split(net2)
        h1[rows, :] = hi
        l1[rows, :] = lo

    @pl.when(stage == 2)
    def _s2():
        g = _gather(jx_ref[blk, :], h1[...], l1[...])
        h = _relu(_dot(g, W(4)) + bias(9)[None, :])
        net3 = sB[rows, :] + _dot(h, W(5)) + bias(10)[None, :]
        sC[rows, :] = net3
        e1 = jnp.exp(_dot(net3, W(6)) + bias(11)[None, :])
        sA[rows, :] = e1
        hi, lo = _split(e1)
        h0[rows, :] = hi
        l0[rows, :] = lo
        sE[rows, :] = _dot(net3, W(7)) + bias(12)[None, :]

    @pl.when(stage == 3)
    def _s3():
        q = jnp.clip(kkf_ref[0, :], 0, N - 1)
        denom = _segsum(blk, q, h0[...], l0[...])
        z1 = sE[rows, :] * (sA[rows, :] / jnp.maximum(denom, 1e-6))
        hi, lo = _split(z1)
        h1[rows, :] = hi
        l1[rows, :] = lo

    @pl.when(stage == 4)
    def _s4():
        q = jnp.clip(kkf_ref[0, :], 0, N - 1)
        y = _segsum(blk, q, h1[...], l1[...])
        net4 = sC[rows, :] + _dot(y, W(8)) + bias(13)[None, :]
        sD[rows, :] = net4
        e2 = jnp.exp(_dot(net4, W(9)) + bias(14)[None, :])
        sB[rows, :] = e2
        hi, lo = _split(e2)
        h0[rows, :] = hi
        l0[rows, :] = lo
        sE[rows, :] = _dot(net4, W(10)) + bias(15)[None, :]

    @pl.when(stage == 5)
    def _s5():
        q2 = jnp.clip(iif_ref[0, :] * 12345 + jjf_ref[0, :], 0, N - 1)
        denom = _segsum(blk, q2, h0[...], l0[...])
        z2 = sE[rows, :] * (sB[rows, :] / jnp.maximum(denom, 1e-6))
        hi, lo = _split(z2)
        h1[rows, :] = hi
        l1[rows, :] = lo

    @pl.when(stage == 6)
    def _s6():
        q2 = jnp.clip(iif_ref[0, :] * 12345 + jjf_ref[0, :], 0, N - 1)
        y = _segsum(blk, q2, h1[...], l1[...])
        x = sD[rows, :] + _dot(y, W(11)) + bias(16)[None, :]
        x = _ln(x, bias(17), bias(18))
        gate = jax.nn.sigmoid(_dot(x, W(12)) + bias(19)[None, :])
        res = _dot(_relu(_dot(x, W(13)) + bias(20)[None, :]), W(14)) + bias(21)[None, :]
        x = x + gate * res
        x = _ln(x, bias(22), bias(23))
        gate = jax.nn.sigmoid(_dot(x, W(15)) + bias(24)[None, :])
        res = _dot(_relu(_dot(x, W(16)) + bias(25)[None, :]), W(17)) + bias(26)[None, :]
        x = x + gate * res
        net5_ref[...] = x
        h = _dot(_relu(x), hw_ref[...]) + bias(27)[:128][None, :]
        cid = jax.lax.broadcasted_iota(jnp.int32, (R, 128), 1)
        head_ref[...] = jnp.where((cid >= 2) & (cid < 4), jax.nn.sigmoid(h), h)


def _shape(s, dtype=F32):
    return jax.ShapeDtypeStruct(s, dtype)


def kernel(net, inp, corr, ii, jj, kk, params):
    p = params
    net2d = jnp.transpose(net[0, :, :, 0], (1, 0))
    inp2d = jnp.transpose(inp[0, :, :, 0], (1, 0))
    corr2d = jnp.transpose(corr[0, :, :, 0], (1, 0))
    iif = ii.astype(jnp.int32).reshape(1, N)
    jjf = jj.astype(jnp.int32).reshape(1, N)
    kkf = kk.astype(jnp.int32).reshape(1, N)
    kkb = kkf.reshape(NB, 1, R)
    jjb = jjf.reshape(NB, 1, R)

    def wT(q):
        return jnp.transpose(q["w"], (1, 0))

    wstk = jnp.stack([
        wT(p["corr1"]), wT(p["corr2"]),
        wT(p["c1"]["l1"]), wT(p["c1"]["l2"]),
        wT(p["c2"]["l1"]), wT(p["c2"]["l2"]),
        wT(p["agg_kk"]["g"]), wT(p["agg_kk"]["f"]), wT(p["agg_kk"]["h"]),
        wT(p["agg_ij"]["g"]), wT(p["agg_ij"]["f"]), wT(p["agg_ij"]["h"]),
        wT(p["gru_gr1"]["gate"]), wT(p["gru_gr1"]["r1"]), wT(p["gru_gr1"]["r2"]),
        wT(p["gru_gr2"]["gate"]), wT(p["gru_gr2"]["r1"]), wT(p["gru_gr2"]["r2"]),
    ])
    hw = jnp.zeros((D, 128), F32)
    hw = hw.at[:, 0:2].set(wT(p["d"]))
    hw = hw.at[:, 2:4].set(wT(p["w"]))
    hb = jnp.zeros((D,), F32)
    hb = hb.at[0:2].set(p["d"]["b"])
    hb = hb.at[2:4].set(p["w"]["b"])
    bias = jnp.stack([
        p["corr0"]["b"], p["corr1"]["b"], p["corr_ln"]["w"], p["corr_ln"]["b"],
        p["corr2"]["b"], p["norm"]["w"], p["norm"]["b"],
        p["c1"]["l1"]["b"], p["c1"]["l2"]["b"],
        p["c2"]["l1"]["b"], p["c2"]["l2"]["b"],
        p["agg_kk"]["g"]["b"], p["agg_kk"]["f"]["b"], p["agg_kk"]["h"]["b"],
        p["agg_ij"]["g"]["b"], p["agg_ij"]["f"]["b"], p["agg_ij"]["h"]["b"],
        p["gru_ln1"]["w"], p["gru_ln1"]["b"],
        p["gru_gr1"]["gate"]["b"], p["gru_gr1"]["r1"]["b"], p["gru_gr1"]["r2"]["b"],
        p["gru_ln2"]["w"], p["gru_ln2"]["b"],
        p["gru_gr2"]["gate"]["b"], p["gru_gr2"]["r1"]["b"], p["gru_gr2"]["r2"]["b"],
        hb,
    ])

    def _in_blk(i):
        return (jnp.minimum(i, NB - 1), 0)

    def _in_blk3(i):
        return (jnp.minimum(i, NB - 1), 0, 0)

    def _out_blk(i):
        return (jnp.maximum(i - 6 * NB, 0), 0)

    def _c0(i):
        return (0, 0)

    def _c03(i):
        return (0, 0, 0)

    grid = (7 * NB,)
    net5, head = pl.pallas_call(
        _mega,
        grid=grid,
        in_specs=[
            pl.BlockSpec((R, CD), _in_blk),
            pl.BlockSpec((R, D), _in_blk),
            pl.BlockSpec((R, D), _in_blk),
            pl.BlockSpec((1, N), _c0),
            pl.BlockSpec((1, N), _c0),
            pl.BlockSpec((1, N), _c0),
            pl.BlockSpec((1, 1, R), _in_blk3),
            pl.BlockSpec((1, 1, R), _in_blk3),
            pl.BlockSpec((CD, D), _c0),
            pl.BlockSpec((18, D, D), _c03),
            pl.BlockSpec((D, 128), _c0),
            pl.BlockSpec((28, D), _c0),
        ],
        out_specs=[
            pl.BlockSpec((R, D), _out_blk),
            pl.BlockSpec((R, 128), _out_blk),
        ],
        out_shape=[_shape((N, D)), _shape((N, 128))],
        scratch_shapes=[
            pltpu.VMEM((N, D), F32), pltpu.VMEM((N, D), F32),
            pltpu.VMEM((N, D), F32), pltpu.VMEM((N, D), F32),
            pltpu.VMEM((N, D), F32),
            pltpu.VMEM((N, D), BF16), pltpu.VMEM((N, D), BF16),
            pltpu.VMEM((N, D), BF16), pltpu.VMEM((N, D), BF16),
            pltpu.VMEM((NB, R), jnp.int32), pltpu.VMEM((NB, R), jnp.int32),
        ],
    )(corr2d, net2d, inp2d, kkf, jjf, iif, kkb, jjb,
      wT(p["corr0"]), wstk, hw, bias)

    net_out = net5.reshape(1, N, D)
    d_out = head[:, 0:2].reshape(1, N, 2)
    w_out = head[:, 2:4].reshape(1, N, 2)
    return net_out, d_out, w_out


# speedup vs baseline: 1.1011x; 1.1011x over previous
import jax
import jax.numpy as jnp
from jax.experimental import pallas as pl
from jax.experimental.pallas import tpu as pltpu

N = 2048
D = 384
CD = 2 * 49 * 3 * 3
R = 256
NB = N // R
F32 = jnp.float32
BF16 = jnp.bfloat16


def _dot(a, b):
    return jax.lax.dot_general(a.astype(BF16), b, (((1,), (0,)), ((), ())),
                               preferred_element_type=F32)


def _ln(x, w, b, eps=1e-3):
    m = jnp.mean(x, axis=-1, keepdims=True)
    v = jnp.mean((x - m) ** 2, axis=-1, keepdims=True)
    return (x - m) / jnp.sqrt(v + eps) * w[None, :] + b[None, :]


def _relu(x):
    return jnp.maximum(x, 0.0)


def _san(x):
    return jnp.where(jnp.isnan(x), 0.0, jnp.clip(x, -3e38, 3e38)).astype(BF16)


def _gather(idx, vf):
    colid = jax.lax.broadcasted_iota(jnp.int32, (R, N), 1)
    m = (idx[:, None] == colid).astype(BF16)
    return jax.lax.dot_general(m, vf, (((1,), (0,)), ((), ())),
                               preferred_element_type=F32)


def _segsum(blk, q, vf):
    rowid = blk * R + jax.lax.broadcasted_iota(jnp.int32, (R, N), 0)
    m = (q[None, :] == rowid).astype(BF16)
    return jax.lax.dot_general(m, vf, (((1,), (0,)), ((), ())),
                               preferred_element_type=F32)


def _mega(corr_ref, netin_ref, inpin_ref, kkf_ref, jjf_ref, iif_ref,
          kkb_ref, jjb_ref, w0_ref, wstk_ref, hw_ref, bias_ref,
          net5_ref, head_ref,
          sA, sB, sC, sD, sE, b0, b1, ix_ref, jx_ref):
    s = pl.program_id(0)
    stage = s // NB
    blk = s % NB
    start = pl.multiple_of(blk * R, R)
    rows = pl.ds(start, R)

    def W(k):
        return wstk_ref[k]

    def bias(k):
        return bias_ref[k, :]

    @pl.when(stage == 0)
    def _s0():
        c = _relu(_dot(corr_ref[...], w0_ref[...]) + bias(0)[None, :])
        c = _dot(c, W(0)) + bias(1)[None, :]
        c = _relu(_ln(c, bias(2), bias(3)))
        c = _dot(c, W(1)) + bias(4)[None, :]
        x = netin_ref[...] + inpin_ref[...] + c
        net1 = _ln(x, bias(5), bias(6))
        sA[rows, :] = net1
        b0[rows, :] = net1.astype(BF16)
        kkf = kkf_ref[0, :]
        jjf = jjf_ref[0, :]
        kkb = kkb_ref[0, 0, :]
        jjb = jjb_ref[0, 0, :]
        mask = kkf[None, :] == kkb[:, None]
        colid = jax.lax.broadcasted_iota(jnp.int32, (R, N), 1)
        jjfc = jnp.broadcast_to(jjf[None, :], (R, N))
        jjbc = jjb[:, None]
        prev_vals = jnp.where(mask & (jjfc < jjbc), jjfc, 0)
        pm = jnp.max(prev_vals, axis=1, keepdims=True)
        ixv = jnp.min(jnp.where(prev_vals == pm, colid, N), axis=1)
        next_vals = jnp.where(mask & (jjfc > jjbc), jjfc, N)
        nm = jnp.min(next_vals, axis=1, keepdims=True)
        jxv = jnp.min(jnp.where(next_vals == nm, colid, N), axis=1)
        ix_ref[blk, :] = jnp.clip(ixv, 0, N - 1)
        jx_ref[blk, :] = jnp.clip(jxv, 0, N - 1)

    @pl.when(stage == 1)
    def _s1():
        g = _gather(ix_ref[blk, :], b0[...])
        h = _relu(_dot(g, W(2)) + bias(7)[None, :])
        net2 = sA[rows, :] + _dot(h, W(3)) + bias(8)[None, :]
        sB[rows, :] = net2
        b1[rows, :] = net2.astype(BF16)

    @pl.when(stage == 2)
    def _s2():
        g = _gather(jx_ref[blk, :], b1[...])
        h = _relu(_dot(g, W(4)) + bias(9)[None, :])
        net3 = sB[rows, :] + _dot(h, W(5)) + bias(10)[None, :]
        sC[rows, :] = net3
        e1 = jnp.exp(_dot(net3, W(6)) + bias(11)[None, :])
        sA[rows, :] = e1
        b0[rows, :] = _san(e1)
        sE[rows, :] = _dot(net3, W(7)) + bias(12)[None, :]

    @pl.when(stage == 3)
    def _s3():
        q = jnp.clip(kkf_ref[0, :], 0, N - 1)
        denom = _segsum(blk, q, b0[...])
        z1 = sE[rows, :] * (sA[rows, :] / jnp.maximum(denom, 1e-6))
        b1[rows, :] = _san(z1)

    @pl.when(stage == 4)
    def _s4():
        q = jnp.clip(kkf_ref[0, :], 0, N - 1)
        y = _segsum(blk, q, b1[...])
        net4 = sC[rows, :] + _dot(y, W(8)) + bias(13)[None, :]
        sD[rows, :] = net4
        e2 = jnp.exp(_dot(net4, W(9)) + bias(14)[None, :])
        sB[rows, :] = e2
        b0[rows, :] = _san(e2)
        sE[rows, :] = _dot(net4, W(10)) + bias(15)[None, :]

    @pl.when(stage == 5)
    def _s5():
        q2 = jnp.clip(iif_ref[0, :] * 12345 + jjf_ref[0, :], 0, N - 1)
        denom = _segsum(blk, q2, b0[...])
        z2 = sE[rows, :] * (sB[rows, :] / jnp.maximum(denom, 1e-6))
        b1[rows, :] = _san(z2)

    @pl.when(stage == 6)
    def _s6():
        q2 = jnp.clip(iif_ref[0, :] * 12345 + jjf_ref[0, :], 0, N - 1)
        y = _segsum(blk, q2, b1[...])
        x = sD[rows, :] + _dot(y, W(11)) + bias(16)[None, :]
        x = _ln(x, bias(17), bias(18))
        gate = jax.nn.sigmoid(_dot(x, W(12)) + bias(19)[None, :])
        res = _dot(_relu(_dot(x, W(13)) + bias(20)[None, :]), W(14)) + bias(21)[None, :]
        x = x + gate * res
        x = _ln(x, bias(22), bias(23))
        gate = jax.nn.sigmoid(_dot(x, W(15)) + bias(24)[None, :])
        res = _dot(_relu(_dot(x, W(16)) + bias(25)[None, :]), W(17)) + bias(26)[None, :]
        x = x + gate * res
        net5_ref[...] = x
        h = _dot(_relu(x), hw_ref[...]) + bias(27)[:128][None, :]
        cid = jax.lax.broadcasted_iota(jnp.int32, (R, 128), 1)
        head_ref[...] = jnp.where((cid >= 2) & (cid < 4), jax.nn.sigmoid(h), h)


def _shape(s, dtype=F32):
    return jax.ShapeDtypeStruct(s, dtype)


def kernel(net, inp, corr, ii, jj, kk, params):
    p = params
    net2d = jnp.transpose(net[0, :, :, 0], (1, 0))
    inp2d = jnp.transpose(inp[0, :, :, 0], (1, 0))
    corr2d = jnp.transpose(corr[0, :, :, 0], (1, 0))
    iif = ii.astype(jnp.int32).reshape(1, N)
    jjf = jj.astype(jnp.int32).reshape(1, N)
    kkf = kk.astype(jnp.int32).reshape(1, N)
    kkb = kkf.reshape(NB, 1, R)
    jjb = jjf.reshape(NB, 1, R)

    def wT(q):
        return jnp.transpose(q["w"], (1, 0)).astype(BF16)

    wstk = jnp.stack([
        wT(p["corr1"]), wT(p["corr2"]),
        wT(p["c1"]["l1"]), wT(p["c1"]["l2"]),
        wT(p["c2"]["l1"]), wT(p["c2"]["l2"]),
        wT(p["agg_kk"]["g"]), wT(p["agg_kk"]["f"]), wT(p["agg_kk"]["h"]),
        wT(p["agg_ij"]["g"]), wT(p["agg_ij"]["f"]), wT(p["agg_ij"]["h"]),
        wT(p["gru_gr1"]["gate"]), wT(p["gru_gr1"]["r1"]), wT(p["gru_gr1"]["r2"]),
        wT(p["gru_gr2"]["gate"]), wT(p["gru_gr2"]["r1"]), wT(p["gru_gr2"]["r2"]),
    ])
    hw = jnp.zeros((D, 128), F32)
    hw = hw.at[:, 0:2].set(jnp.transpose(p["d"]["w"], (1, 0)))
    hw = hw.at[:, 2:4].set(jnp.transpose(p["w"]["w"], (1, 0)))
    hw = hw.astype(BF16)
    hb = jnp.zeros((D,), F32)
    hb = hb.at[0:2].set(p["d"]["b"])
    hb = hb.at[2:4].set(p["w"]["b"])
    bias = jnp.stack([
        p["corr0"]["b"], p["corr1"]["b"], p["corr_ln"]["w"], p["corr_ln"]["b"],
        p["corr2"]["b"], p["norm"]["w"], p["norm"]["b"],
        p["c1"]["l1"]["b"], p["c1"]["l2"]["b"],
        p["c2"]["l1"]["b"], p["c2"]["l2"]["b"],
        p["agg_kk"]["g"]["b"], p["agg_kk"]["f"]["b"], p["agg_kk"]["h"]["b"],
        p["agg_ij"]["g"]["b"], p["agg_ij"]["f"]["b"], p["agg_ij"]["h"]["b"],
        p["gru_ln1"]["w"], p["gru_ln1"]["b"],
        p["gru_gr1"]["gate"]["b"], p["gru_gr1"]["r1"]["b"], p["gru_gr1"]["r2"]["b"],
        p["gru_ln2"]["w"], p["gru_ln2"]["b"],
        p["gru_gr2"]["gate"]["b"], p["gru_gr2"]["r1"]["b"], p["gru_gr2"]["r2"]["b"],
        hb,
    ])

    def _in_blk(i):
        return (jnp.minimum(i, NB - 1), 0)

    def _in_blk3(i):
        return (jnp.minimum(i, NB - 1), 0, 0)

    def _out_blk(i):
        return (jnp.maximum(i - 6 * NB, 0), 0)

    def _c0(i):
        return (0, 0)

    def _c03(i):
        return (0, 0, 0)

    grid = (7 * NB,)
    net5, head = pl.pallas_call(
        _mega,
        grid=grid,
        in_specs=[
            pl.BlockSpec((R, CD), _in_blk),
            pl.BlockSpec((R, D), _in_blk),
            pl.BlockSpec((R, D), _in_blk),
            pl.BlockSpec((1, N), _c0),
            pl.BlockSpec((1, N), _c0),
            pl.BlockSpec((1, N), _c0),
            pl.BlockSpec((1, 1, R), _in_blk3),
            pl.BlockSpec((1, 1, R), _in_blk3),
            pl.BlockSpec((CD, D), _c0),
            pl.BlockSpec((18, D, D), _c03),
            pl.BlockSpec((D, 128), _c0),
            pl.BlockSpec((28, D), _c0),
        ],
        out_specs=[
            pl.BlockSpec((R, D), _out_blk),
            pl.BlockSpec((R, 128), _out_blk),
        ],
        out_shape=[_shape((N, D)), _shape((N, 128))],
        scratch_shapes=[
            pltpu.VMEM((N, D), F32), pltpu.VMEM((N, D), F32),
            pltpu.VMEM((N, D), F32), pltpu.VMEM((N, D), F32),
            pltpu.VMEM((N, D), F32),
            pltpu.VMEM((N, D), BF16), pltpu.VMEM((N, D), BF16),
            pltpu.VMEM((NB, R), jnp.int32), pltpu.VMEM((NB, R), jnp.int32),
        ],
    )(corr2d, net2d, inp2d, kkf, jjf, iif, kkb, jjb,
      jnp.transpose(p["corr0"]["w"], (1, 0)).astype(BF16), wstk, hw, bias)

    net_out = net5.reshape(1, N, D)
    d_out = head[:, 0:2].reshape(1, N, 2)
    w_out = head[:, 2:4].reshape(1, N, 2)
    return net_out, d_out, w_out


# speedup vs baseline: 1.2629x; 1.1470x over previous
import jax
import jax.numpy as jnp
from jax.experimental import pallas as pl
from jax.experimental.pallas import tpu as pltpu

N = 2048
D = 384
CD = 2 * 49 * 3 * 3
R = 512
NB = N // R
F32 = jnp.float32
BF16 = jnp.bfloat16


def _dot(a, b):
    return jax.lax.dot_general(a.astype(BF16), b, (((1,), (0,)), ((), ())),
                               preferred_element_type=F32)


def _ln(x, w, b, eps=1e-3):
    m = jnp.mean(x, axis=-1, keepdims=True)
    v = jnp.mean((x - m) ** 2, axis=-1, keepdims=True)
    return (x - m) / jnp.sqrt(v + eps) * w[None, :] + b[None, :]


def _relu(x):
    return jnp.maximum(x, 0.0)


def _san(x):
    return jnp.where(jnp.isnan(x), 0.0, jnp.clip(x, -3e38, 3e38)).astype(BF16)


def _gather(idx, vf):
    colid = jax.lax.broadcasted_iota(jnp.int32, (R, N), 1)
    m = (idx[:, None] == colid).astype(BF16)
    return jax.lax.dot_general(m, vf, (((1,), (0,)), ((), ())),
                               preferred_element_type=F32)


def _seg_mask(blk, q):
    rowid = blk * R + jax.lax.broadcasted_iota(jnp.int32, (R, N), 0)
    return (q[None, :] == rowid).astype(BF16)


def _mdot(m, vf):
    return jax.lax.dot_general(m, vf, (((1,), (0,)), ((), ())),
                               preferred_element_type=F32)


def _mega(corr_ref, netin_ref, inpin_ref, kkf_ref, jjf_ref, iif_ref,
          kkb_ref, jjb_ref, w0_ref, wstk_ref, hw_ref, bias_ref,
          net5_ref, head_ref,
          sA, sB, sC, sD, sE, b0, b1, mscr, ix_ref, jx_ref):
    s = pl.program_id(0)
    stage = s // NB
    blk = s % NB
    start = pl.multiple_of(blk * R, R)
    rows = pl.ds(start, R)

    def W(k):
        return wstk_ref[k]

    def bias(k):
        return bias_ref[k, :]

    @pl.when(stage == 0)
    def _s0():
        c = _relu(_dot(corr_ref[...], w0_ref[...]) + bias(0)[None, :])
        c = _dot(c, W(0)) + bias(1)[None, :]
        c = _relu(_ln(c, bias(2), bias(3)))
        c = _dot(c, W(1)) + bias(4)[None, :]
        x = netin_ref[...] + inpin_ref[...] + c
        net1 = _ln(x, bias(5), bias(6))
        sA[rows, :] = net1
        b0[rows, :] = net1.astype(BF16)
        kkf = kkf_ref[0, :]
        jjf = jjf_ref[0, :]
        kkb = kkb_ref[0, 0, :]
        jjb = jjb_ref[0, 0, :]
        mask = kkf[None, :] == kkb[:, None]
        colid = jax.lax.broadcasted_iota(jnp.int32, (R, N), 1)
        jjfc = jnp.broadcast_to(jjf[None, :], (R, N))
        jjbc = jjb[:, None]
        pkey = jjfc * 4096 + (4095 - colid)
        kmax = jnp.max(jnp.where(mask & (jjfc < jjbc), pkey, -1), axis=1)
        ixv = jnp.where(kmax >= 4096, 4095 - (kmax & 4095), 0)
        nkey = jjfc * 4096 + colid
        big = jnp.int32(2 ** 30)
        kmin = jnp.min(jnp.where(mask & (jjfc > jjbc), nkey, big), axis=1)
        jxv = jnp.where(kmin < big, kmin & 4095, 0)
        ix_ref[blk, :] = ixv
        jx_ref[blk, :] = jxv

    @pl.when(stage == 1)
    def _s1():
        g = _gather(ix_ref[blk, :], b0[...])
        h = _relu(_dot(g, W(2)) + bias(7)[None, :])
        net2 = sA[rows, :] + _dot(h, W(3)) + bias(8)[None, :]
        sB[rows, :] = net2
        b1[rows, :] = net2.astype(BF16)

    @pl.when(stage == 2)
    def _s2():
        g = _gather(jx_ref[blk, :], b1[...])
        h = _relu(_dot(g, W(4)) + bias(9)[None, :])
        net3 = sB[rows, :] + _dot(h, W(5)) + bias(10)[None, :]
        sC[rows, :] = net3
        e1 = jnp.exp(_dot(net3, W(6)) + bias(11)[None, :])
        sA[rows, :] = e1
        b0[rows, :] = _san(e1)
        sE[rows, :] = _dot(net3, W(7)) + bias(12)[None, :]

    @pl.when(stage == 3)
    def _s3():
        q = jnp.clip(kkf_ref[0, :], 0, N - 1)
        m = _seg_mask(blk, q)
        mscr[rows, :] = m
        denom = _mdot(m, b0[...])
        z1 = sE[rows, :] * (sA[rows, :] / jnp.maximum(denom, 1e-6))
        b1[rows, :] = _san(z1)

    @pl.when(stage == 4)
    def _s4():
        y = _mdot(mscr[rows, :], b1[...])
        net4 = sC[rows, :] + _dot(y, W(8)) + bias(13)[None, :]
        sD[rows, :] = net4
        e2 = jnp.exp(_dot(net4, W(9)) + bias(14)[None, :])
        sB[rows, :] = e2
        b0[rows, :] = _san(e2)
        sE[rows, :] = _dot(net4, W(10)) + bias(15)[None, :]

    @pl.when(stage == 5)
    def _s5():
        q2 = jnp.clip(iif_ref[0, :] * 12345 + jjf_ref[0, :], 0, N - 1)
        m = _seg_mask(blk, q2)
        mscr[rows, :] = m
        denom = _mdot(m, b0[...])
        z2 = sE[rows, :] * (sB[rows, :] / jnp.maximum(denom, 1e-6))
        b1[rows, :] = _san(z2)

    @pl.when(stage == 6)
    def _s6():
        y = _mdot(mscr[rows, :], b1[...])
        x = sD[rows, :] + _dot(y, W(11)) + bias(16)[None, :]
        x = _ln(x, bias(17), bias(18))
        gate = jax.nn.sigmoid(_dot(x, W(12)) + bias(19)[None, :])
        res = _dot(_relu(_dot(x, W(13)) + bias(20)[None, :]), W(14)) + bias(21)[None, :]
        x = x + gate * res
        x = _ln(x, bias(22), bias(23))
        gate = jax.nn.sigmoid(_dot(x, W(15)) + bias(24)[None, :])
        res = _dot(_relu(_dot(x, W(16)) + bias(25)[None, :]), W(17)) + bias(26)[None, :]
        x = x + gate * res
        net5_ref[...] = x
        h = _dot(_relu(x), hw_ref[...]) + bias(27)[:128][None, :]
        cid = jax.lax.broadcasted_iota(jnp.int32, (R, 128), 1)
        head_ref[...] = jnp.where((cid >= 2) & (cid < 4), jax.nn.sigmoid(h), h)


def _shape(s, dtype=F32):
    return jax.ShapeDtypeStruct(s, dtype)


def kernel(net, inp, corr, ii, jj, kk, params):
    p = params
    net2d = jnp.transpose(net[0, :, :, 0], (1, 0))
    inp2d = jnp.transpose(inp[0, :, :, 0], (1, 0))
    corr2d = jnp.transpose(corr[0, :, :, 0], (1, 0))
    iif = ii.astype(jnp.int32).reshape(1, N)
    jjf = jj.astype(jnp.int32).reshape(1, N)
    kkf = kk.astype(jnp.int32).reshape(1, N)
    kkb = kkf.reshape(NB, 1, R)
    jjb = jjf.reshape(NB, 1, R)

    def wT(q):
        return jnp.transpose(q["w"], (1, 0)).astype(BF16)

    wstk = jnp.stack([
        wT(p["corr1"]), wT(p["corr2"]),
        wT(p["c1"]["l1"]), wT(p["c1"]["l2"]),
        wT(p["c2"]["l1"]), wT(p["c2"]["l2"]),
        wT(p["agg_kk"]["g"]), wT(p["agg_kk"]["f"]), wT(p["agg_kk"]["h"]),
        wT(p["agg_ij"]["g"]), wT(p["agg_ij"]["f"]), wT(p["agg_ij"]["h"]),
        wT(p["gru_gr1"]["gate"]), wT(p["gru_gr1"]["r1"]), wT(p["gru_gr1"]["r2"]),
        wT(p["gru_gr2"]["gate"]), wT(p["gru_gr2"]["r1"]), wT(p["gru_gr2"]["r2"]),
    ])
    hw = jnp.zeros((D, 128), F32)
    hw = hw.at[:, 0:2].set(jnp.transpose(p["d"]["w"], (1, 0)))
    hw = hw.at[:, 2:4].set(jnp.transpose(p["w"]["w"], (1, 0)))
    hw = hw.astype(BF16)
    hb = jnp.zeros((D,), F32)
    hb = hb.at[0:2].set(p["d"]["b"])
    hb = hb.at[2:4].set(p["w"]["b"])
    bias = jnp.stack([
        p["corr0"]["b"], p["corr1"]["b"], p["corr_ln"]["w"], p["corr_ln"]["b"],
        p["corr2"]["b"], p["norm"]["w"], p["norm"]["b"],
        p["c1"]["l1"]["b"], p["c1"]["l2"]["b"],
        p["c2"]["l1"]["b"], p["c2"]["l2"]["b"],
        p["agg_kk"]["g"]["b"], p["agg_kk"]["f"]["b"], p["agg_kk"]["h"]["b"],
        p["agg_ij"]["g"]["b"], p["agg_ij"]["f"]["b"], p["agg_ij"]["h"]["b"],
        p["gru_ln1"]["w"], p["gru_ln1"]["b"],
        p["gru_gr1"]["gate"]["b"], p["gru_gr1"]["r1"]["b"], p["gru_gr1"]["r2"]["b"],
        p["gru_ln2"]["w"], p["gru_ln2"]["b"],
        p["gru_gr2"]["gate"]["b"], p["gru_gr2"]["r1"]["b"], p["gru_gr2"]["r2"]["b"],
        hb,
    ])

    def _in_blk(i):
        return (jnp.minimum(i, NB - 1), 0)

    def _in_blk3(i):
        return (jnp.minimum(i, NB - 1), 0, 0)

    def _out_blk(i):
        return (jnp.maximum(i - 6 * NB, 0), 0)

    def _c0(i):
        return (0, 0)

    def _c03(i):
        return (0, 0, 0)

    grid = (7 * NB,)
    net5, head = pl.pallas_call(
        _mega,
        grid=grid,
        in_specs=[
            pl.BlockSpec((R, CD), _in_blk),
            pl.BlockSpec((R, D), _in_blk),
            pl.BlockSpec((R, D), _in_blk),
            pl.BlockSpec((1, N), _c0),
            pl.BlockSpec((1, N), _c0),
            pl.BlockSpec((1, N), _c0),
            pl.BlockSpec((1, 1, R), _in_blk3),
            pl.BlockSpec((1, 1, R), _in_blk3),
            pl.BlockSpec((CD, D), _c0),
            pl.BlockSpec((18, D, D), _c03),
            pl.BlockSpec((D, 128), _c0),
            pl.BlockSpec((28, D), _c0),
        ],
        out_specs=[
            pl.BlockSpec((R, D), _out_blk),
            pl.BlockSpec((R, 128), _out_blk),
        ],
        out_shape=[_shape((N, D)), _shape((N, 128))],
        scratch_shapes=[
            pltpu.VMEM((N, D), F32), pltpu.VMEM((N, D), F32),
            pltpu.VMEM((N, D), F32), pltpu.VMEM((N, D), F32),
            pltpu.VMEM((N, D), F32),
            pltpu.VMEM((N, D), BF16), pltpu.VMEM((N, D), BF16),
            pltpu.VMEM((N, N), BF16),
            pltpu.VMEM((NB, R), jnp.int32), pltpu.VMEM((NB, R), jnp.int32),
        ],
    )(corr2d, net2d, inp2d, kkf, jjf, iif, kkb, jjb,
      jnp.transpose(p["corr0"]["w"], (1, 0)).astype(BF16), wstk, hw, bias)

    net_out = net5.reshape(1, N, D)
    d_out = head[:, 0:2].reshape(1, N, 2)
    w_out = head[:, 2:4].reshape(1, N, 2)
    return net_out, d_out, w_out


# speedup vs baseline: 1.2892x; 1.0208x over previous
import jax
import jax.numpy as jnp
from jax.experimental import pallas as pl
from jax.experimental.pallas import tpu as pltpu

N = 2048
D = 384
CD = 2 * 49 * 3 * 3
R = 1024
NB = N // R
F32 = jnp.float32
BF16 = jnp.bfloat16


def _dot(a, b):
    return jax.lax.dot_general(a.astype(BF16), b, (((1,), (0,)), ((), ())),
                               preferred_element_type=F32)


def _ln(x, w, b, eps=1e-3):
    m = jnp.mean(x, axis=-1, keepdims=True)
    v = jnp.mean((x - m) ** 2, axis=-1, keepdims=True)
    return (x - m) / jnp.sqrt(v + eps) * w[None, :] + b[None, :]


def _relu(x):
    return jnp.maximum(x, 0.0)


def _san(x):
    return jnp.where(jnp.isnan(x), 0.0, jnp.clip(x, -3e38, 3e38)).astype(BF16)


def _gather(idx, vf):
    colid = jax.lax.broadcasted_iota(jnp.int32, (R, N), 1)
    m = (idx[:, None] == colid).astype(BF16)
    return jax.lax.dot_general(m, vf, (((1,), (0,)), ((), ())),
                               preferred_element_type=F32)


def _seg_mask(blk, q):
    rowid = blk * R + jax.lax.broadcasted_iota(jnp.int32, (R, N), 0)
    return (q[None, :] == rowid).astype(BF16)


def _mdot(m, vf):
    return jax.lax.dot_general(m, vf, (((1,), (0,)), ((), ())),
                               preferred_element_type=F32)


def _mega(corr_ref, netin_ref, inpin_ref, kkf_ref, jjf_ref, iif_ref,
          kkb_ref, jjb_ref, w0_ref, wstk_ref, hw_ref, bias_ref,
          net5_ref, head_ref,
          sA, sB, sC, sD, sE, b0, b1, mscr, ix_ref, jx_ref):
    s = pl.program_id(0)
    stage = s // NB
    blk = s % NB
    start = pl.multiple_of(blk * R, R)
    rows = pl.ds(start, R)

    def W(k):
        return wstk_ref[k]

    def bias(k):
        return bias_ref[k, :]

    @pl.when(stage == 0)
    def _s0():
        c = _relu(_dot(corr_ref[...], w0_ref[...]) + bias(0)[None, :])
        c = _dot(c, W(0)) + bias(1)[None, :]
        c = _relu(_ln(c, bias(2), bias(3)))
        c = _dot(c, W(1)) + bias(4)[None, :]
        x = netin_ref[...] + inpin_ref[...] + c
        net1 = _ln(x, bias(5), bias(6))
        sA[rows, :] = net1
        b0[rows, :] = net1.astype(BF16)
        kkf = kkf_ref[0, :]
        jjf = jjf_ref[0, :]
        kkb = kkb_ref[0, 0, :]
        jjb = jjb_ref[0, 0, :]
        mask = kkf[None, :] == kkb[:, None]
        colid = jax.lax.broadcasted_iota(jnp.int32, (R, N), 1)
        jjfc = jnp.broadcast_to(jjf[None, :], (R, N))
        jjbc = jjb[:, None]
        pkey = jjfc * 4096 + (4095 - colid)
        kmax = jnp.max(jnp.where(mask & (jjfc < jjbc), pkey, -1), axis=1)
        ixv = jnp.where(kmax >= 4096, 4095 - (kmax & 4095), 0)
        nkey = jjfc * 4096 + colid
        big = jnp.int32(2 ** 30)
        kmin = jnp.min(jnp.where(mask & (jjfc > jjbc), nkey, big), axis=1)
        jxv = jnp.where(kmin < big, kmin & 4095, 0)
        ix_ref[blk, :] = ixv
        jx_ref[blk, :] = jxv

    @pl.when(stage == 1)
    def _s1():
        g = _gather(ix_ref[blk, :], b0[...])
        h = _relu(_dot(g, W(2)) + bias(7)[None, :])
        net2 = sA[rows, :] + _dot(h, W(3)) + bias(8)[None, :]
        sB[rows, :] = net2
        b1[rows, :] = net2.astype(BF16)

    @pl.when(stage == 2)
    def _s2():
        g = _gather(jx_ref[blk, :], b1[...])
        h = _relu(_dot(g, W(4)) + bias(9)[None, :])
        net3 = sB[rows, :] + _dot(h, W(5)) + bias(10)[None, :]
        sC[rows, :] = net3
        e1 = jnp.exp(_dot(net3, W(6)) + bias(11)[None, :])
        sA[rows, :] = e1
        b0[rows, :] = _san(e1)
        sE[rows, :] = _dot(net3, W(7)) + bias(12)[None, :]

    @pl.when(stage == 3)
    def _s3():
        q = jnp.clip(kkf_ref[0, :], 0, N - 1)
        m = _seg_mask(blk, q)
        mscr[rows, :] = m
        denom = _mdot(m, b0[...])
        z1 = sE[rows, :] * (sA[rows, :] / jnp.maximum(denom, 1e-6))
        b1[rows, :] = _san(z1)

    @pl.when(stage == 4)
    def _s4():
        y = _mdot(mscr[rows, :], b1[...])
        net4 = sC[rows, :] + _dot(y, W(8)) + bias(13)[None, :]
        sD[rows, :] = net4
        e2 = jnp.exp(_dot(net4, W(9)) + bias(14)[None, :])
        sB[rows, :] = e2
        b0[rows, :] = _san(e2)
        sE[rows, :] = _dot(net4, W(10)) + bias(15)[None, :]

    @pl.when(stage == 5)
    def _s5():
        q2 = jnp.clip(iif_ref[0, :] * 12345 + jjf_ref[0, :], 0, N - 1)
        m = _seg_mask(blk, q2)
        mscr[rows, :] = m
        denom = _mdot(m, b0[...])
        z2 = sE[rows, :] * (sB[rows, :] / jnp.maximum(denom, 1e-6))
        b1[rows, :] = _san(z2)

    @pl.when(stage == 6)
    def _s6():
        y = _mdot(mscr[rows, :], b1[...])
        x = sD[rows, :] + _dot(y, W(11)) + bias(16)[None, :]
        x = _ln(x, bias(17), bias(18))
        gate = jax.nn.sigmoid(_dot(x, W(12)) + bias(19)[None, :])
        res = _dot(_relu(_dot(x, W(13)) + bias(20)[None, :]), W(14)) + bias(21)[None, :]
        x = x + gate * res
        x = _ln(x, bias(22), bias(23))
        gate = jax.nn.sigmoid(_dot(x, W(15)) + bias(24)[None, :])
        res = _dot(_relu(_dot(x, W(16)) + bias(25)[None, :]), W(17)) + bias(26)[None, :]
        x = x + gate * res
        net5_ref[...] = x
        h = _dot(_relu(x), hw_ref[...]) + bias(27)[:128][None, :]
        cid = jax.lax.broadcasted_iota(jnp.int32, (R, 128), 1)
        head_ref[...] = jnp.where((cid >= 2) & (cid < 4), jax.nn.sigmoid(h), h)


def _shape(s, dtype=F32):
    return jax.ShapeDtypeStruct(s, dtype)


def kernel(net, inp, corr, ii, jj, kk, params):
    p = params
    net2d = jnp.transpose(net[0, :, :, 0], (1, 0))
    inp2d = jnp.transpose(inp[0, :, :, 0], (1, 0))
    corr2d = jnp.transpose(corr[0, :, :, 0], (1, 0))
    iif = ii.astype(jnp.int32).reshape(1, N)
    jjf = jj.astype(jnp.int32).reshape(1, N)
    kkf = kk.astype(jnp.int32).reshape(1, N)
    kkb = kkf.reshape(NB, 1, R)
    jjb = jjf.reshape(NB, 1, R)

    def wT(q):
        return jnp.transpose(q["w"], (1, 0)).astype(BF16)

    wstk = jnp.stack([
        wT(p["corr1"]), wT(p["corr2"]),
        wT(p["c1"]["l1"]), wT(p["c1"]["l2"]),
        wT(p["c2"]["l1"]), wT(p["c2"]["l2"]),
        wT(p["agg_kk"]["g"]), wT(p["agg_kk"]["f"]), wT(p["agg_kk"]["h"]),
        wT(p["agg_ij"]["g"]), wT(p["agg_ij"]["f"]), wT(p["agg_ij"]["h"]),
        wT(p["gru_gr1"]["gate"]), wT(p["gru_gr1"]["r1"]), wT(p["gru_gr1"]["r2"]),
        wT(p["gru_gr2"]["gate"]), wT(p["gru_gr2"]["r1"]), wT(p["gru_gr2"]["r2"]),
    ])
    hw = jnp.zeros((D, 128), F32)
    hw = hw.at[:, 0:2].set(jnp.transpose(p["d"]["w"], (1, 0)))
    hw = hw.at[:, 2:4].set(jnp.transpose(p["w"]["w"], (1, 0)))
    hw = hw.astype(BF16)
    hb = jnp.zeros((D,), F32)
    hb = hb.at[0:2].set(p["d"]["b"])
    hb = hb.at[2:4].set(p["w"]["b"])
    bias = jnp.stack([
        p["corr0"]["b"], p["corr1"]["b"], p["corr_ln"]["w"], p["corr_ln"]["b"],
        p["corr2"]["b"], p["norm"]["w"], p["norm"]["b"],
        p["c1"]["l1"]["b"], p["c1"]["l2"]["b"],
        p["c2"]["l1"]["b"], p["c2"]["l2"]["b"],
        p["agg_kk"]["g"]["b"], p["agg_kk"]["f"]["b"], p["agg_kk"]["h"]["b"],
        p["agg_ij"]["g"]["b"], p["agg_ij"]["f"]["b"], p["agg_ij"]["h"]["b"],
        p["gru_ln1"]["w"], p["gru_ln1"]["b"],
        p["gru_gr1"]["gate"]["b"], p["gru_gr1"]["r1"]["b"], p["gru_gr1"]["r2"]["b"],
        p["gru_ln2"]["w"], p["gru_ln2"]["b"],
        p["gru_gr2"]["gate"]["b"], p["gru_gr2"]["r1"]["b"], p["gru_gr2"]["r2"]["b"],
        hb,
    ])

    def _in_blk(i):
        return (jnp.minimum(i, NB - 1), 0)

    def _in_blk3(i):
        return (jnp.minimum(i, NB - 1), 0, 0)

    def _out_blk(i):
        return (jnp.maximum(i - 6 * NB, 0), 0)

    def _c0(i):
        return (0, 0)

    def _c03(i):
        return (0, 0, 0)

    grid = (7 * NB,)
    net5, head = pl.pallas_call(
        _mega,
        grid=grid,
        in_specs=[
            pl.BlockSpec((R, CD), _in_blk),
            pl.BlockSpec((R, D), _in_blk),
            pl.BlockSpec((R, D), _in_blk),
            pl.BlockSpec((1, N), _c0),
            pl.BlockSpec((1, N), _c0),
            pl.BlockSpec((1, N), _c0),
            pl.BlockSpec((1, 1, R), _in_blk3),
            pl.BlockSpec((1, 1, R), _in_blk3),
            pl.BlockSpec((CD, D), _c0),
            pl.BlockSpec((18, D, D), _c03),
            pl.BlockSpec((D, 128), _c0),
            pl.BlockSpec((28, D), _c0),
        ],
        out_specs=[
            pl.BlockSpec((R, D), _out_blk),
            pl.BlockSpec((R, 128), _out_blk),
        ],
        out_shape=[_shape((N, D)), _shape((N, 128))],
        scratch_shapes=[
            pltpu.VMEM((N, D), F32), pltpu.VMEM((N, D), F32),
            pltpu.VMEM((N, D), F32), pltpu.VMEM((N, D), F32),
            pltpu.VMEM((N, D), F32),
            pltpu.VMEM((N, D), BF16), pltpu.VMEM((N, D), BF16),
            pltpu.VMEM((N, N), BF16),
            pltpu.VMEM((NB, R), jnp.int32), pltpu.VMEM((NB, R), jnp.int32),
        ],
    )(corr2d, net2d, inp2d, kkf, jjf, iif, kkb, jjb,
      jnp.transpose(p["corr0"]["w"], (1, 0)).astype(BF16), wstk, hw, bias)

    net_out = net5.reshape(1, N, D)
    d_out = head[:, 0:2].reshape(1, N, 2)
    w_out = head[:, 2:4].reshape(1, N, 2)
    return net_out, d_out, w_out


# speedup vs baseline: 1.5784x; 1.2244x over previous
import jax
import jax.numpy as jnp
from jax.experimental import pallas as pl
from jax.experimental.pallas import tpu as pltpu

N = 2048
D = 384
CD = 2 * 49 * 3 * 3
R = 1024
NB = N // R
F32 = jnp.float32
BF16 = jnp.bfloat16


def _dot(a, b):
    return jax.lax.dot_general(a.astype(BF16), b, (((1,), (0,)), ((), ())),
                               preferred_element_type=F32)


def _ln(x, w, b, eps=1e-3):
    m = jnp.mean(x, axis=-1, keepdims=True)
    v = jnp.mean((x - m) ** 2, axis=-1, keepdims=True)
    return (x - m) / jnp.sqrt(v + eps) * w[None, :] + b[None, :]


def _relu(x):
    return jnp.maximum(x, 0.0)


def _san(x):
    return jnp.where(jnp.isnan(x), 0.0, jnp.clip(x, -3e38, 3e38)).astype(BF16)


def _gather(idx, vf):
    colid = jax.lax.broadcasted_iota(jnp.int32, (R, N), 1)
    m = (idx[:, None] == colid).astype(BF16)
    return jax.lax.dot_general(m, vf, (((1,), (0,)), ((), ())),
                               preferred_element_type=F32)


def _seg_mask(blk, q):
    rowid = blk * R + jax.lax.broadcasted_iota(jnp.int32, (R, N), 0)
    return (q[None, :] == rowid).astype(BF16)


def _mdot(m, vf):
    return jax.lax.dot_general(m, vf, (((1,), (0,)), ((), ())),
                               preferred_element_type=F32)


def _mega(corr_ref, netin_ref, inpin_ref, kkf_ref, jjf_ref, iif_ref,
          kkb_ref, jjb_ref, w0_ref, wstk_ref, hw_ref, bias_ref,
          net5_ref, head_ref,
          sA, sB, sC, sD, sE, b0, b1, mscr, ix_ref, jx_ref):
    s = pl.program_id(0)
    stage = s // NB
    blk = s % NB
    start = pl.multiple_of(blk * R, R)
    rows = pl.ds(start, R)

    def W(k):
        return wstk_ref[k]

    def bias(k):
        return bias_ref[k, :]

    @pl.when(stage == 0)
    def _s0():
        c = _relu(_dot(corr_ref[...], w0_ref[...]) + bias(0)[None, :])
        c = _dot(c, W(0)) + bias(1)[None, :]
        c = _relu(_ln(c, bias(2), bias(3)))
        c = _dot(c, W(1)) + bias(4)[None, :]
        x = netin_ref[...] + inpin_ref[...] + c
        net1 = _ln(x, bias(5), bias(6))
        sA[rows, :] = net1
        b0[rows, :] = net1.astype(BF16)
        kkf = kkf_ref[0, :]
        jjf = jjf_ref[0, :]
        kkb = kkb_ref[0, 0, :]
        jjb = jjb_ref[0, 0, :]
        mask = kkf[None, :] == kkb[:, None]
        colid = jax.lax.broadcasted_iota(jnp.int32, (R, N), 1)
        jjfc = jnp.broadcast_to(jjf[None, :], (R, N))
        jjbc = jjb[:, None]
        pkey = jjfc * 4096 + (4095 - colid)
        kmax = jnp.max(jnp.where(mask & (jjfc < jjbc), pkey, -1), axis=1)
        ixv = jnp.where(kmax >= 4096, 4095 - (kmax & 4095), 0)
        nkey = jjfc * 4096 + colid
        big = jnp.int32(2 ** 30)
        kmin = jnp.min(jnp.where(mask & (jjfc > jjbc), nkey, big), axis=1)
        jxv = jnp.where(kmin < big, kmin & 4095, 0)
        ix_ref[blk, :] = ixv
        jx_ref[blk, :] = jxv

    @pl.when(stage == 1)
    def _s1():
        g = _gather(ix_ref[blk, :], b0[...])
        h = _relu(_dot(g, W(2)) + bias(7)[None, :])
        net2 = sA[rows, :] + _dot(h, W(3)) + bias(8)[None, :]
        sB[rows, :] = net2
        b1[rows, :] = net2.astype(BF16)

    @pl.when(stage == 2)
    def _s2():
        g = _gather(jx_ref[blk, :], b1[...])
        h = _relu(_dot(g, W(4)) + bias(9)[None, :])
        net3 = sB[rows, :] + _dot(h, W(5)) + bias(10)[None, :]
        sC[rows, :] = net3
        e1 = jnp.exp(_dot(net3, W(6)) + bias(11)[None, :])
        sA[rows, :] = e1
        b0[rows, :] = _san(e1)
        sE[rows, :] = _dot(net3, W(7)) + bias(12)[None, :]

    @pl.when(stage == 3)
    def _s3():
        q = jnp.clip(kkf_ref[0, :], 0, N - 1)
        m = _seg_mask(blk, q)
        mscr[rows, :] = m
        denom = _mdot(m, b0[...])
        z1 = sE[rows, :] * (sA[rows, :] / jnp.maximum(denom, 1e-6))
        b1[rows, :] = _san(z1)

    @pl.when(stage == 4)
    def _s4():
        y = _mdot(mscr[rows, :], b1[...])
        net4 = sC[rows, :] + _dot(y, W(8)) + bias(13)[None, :]
        sD[rows, :] = net4
        e2 = jnp.exp(_dot(net4, W(9)) + bias(14)[None, :])
        sB[rows, :] = e2
        b0[rows, :] = _san(e2)
        sE[rows, :] = _dot(net4, W(10)) + bias(15)[None, :]

    @pl.when(stage == 5)
    def _s5():
        q2 = jnp.clip(iif_ref[0, :] * 12345 + jjf_ref[0, :], 0, N - 1)
        m = _seg_mask(blk, q2)
        mscr[rows, :] = m
        denom = _mdot(m, b0[...])
        z2 = sE[rows, :] * (sB[rows, :] / jnp.maximum(denom, 1e-6))
        b1[rows, :] = _san(z2)

    @pl.when(stage == 6)
    def _s6():
        y = _mdot(mscr[rows, :], b1[...])
        x = sD[rows, :] + _dot(y, W(11)) + bias(16)[None, :]
        x = _ln(x, bias(17), bias(18))
        gate = jax.nn.sigmoid(_dot(x, W(12)) + bias(19)[None, :])
        res = _dot(_relu(_dot(x, W(13)) + bias(20)[None, :]), W(14)) + bias(21)[None, :]
        x = x + gate * res
        x = _ln(x, bias(22), bias(23))
        gate = jax.nn.sigmoid(_dot(x, W(15)) + bias(24)[None, :])
        res = _dot(_relu(_dot(x, W(16)) + bias(25)[None, :]), W(17)) + bias(26)[None, :]
        x = x + gate * res
        net5_ref[...] = x
        h = _dot(_relu(x), hw_ref[...]) + bias(27)[:128][None, :]
        cid = jax.lax.broadcasted_iota(jnp.int32, (R, 128), 1)
        head_ref[...] = jnp.where((cid >= 2) & (cid < 4), jax.nn.sigmoid(h), h)


def _shape(s, dtype=F32):
    return jax.ShapeDtypeStruct(s, dtype)


def kernel(net, inp, corr, ii, jj, kk, params):
    p = params
    net2d = jnp.transpose(net[0, :, :, 0], (1, 0))
    inp2d = jnp.transpose(inp[0, :, :, 0], (1, 0))
    corr2d = jnp.transpose(corr[0, :, :, 0], (1, 0)).astype(BF16)
    iif = ii.astype(jnp.int32).reshape(1, N)
    jjf = jj.astype(jnp.int32).reshape(1, N)
    kkf = kk.astype(jnp.int32).reshape(1, N)
    kkb = kkf.reshape(NB, 1, R)
    jjb = jjf.reshape(NB, 1, R)

    def wT(q):
        return jnp.transpose(q["w"], (1, 0)).astype(BF16)

    wstk = jnp.stack([
        wT(p["corr1"]), wT(p["corr2"]),
        wT(p["c1"]["l1"]), wT(p["c1"]["l2"]),
        wT(p["c2"]["l1"]), wT(p["c2"]["l2"]),
        wT(p["agg_kk"]["g"]), wT(p["agg_kk"]["f"]), wT(p["agg_kk"]["h"]),
        wT(p["agg_ij"]["g"]), wT(p["agg_ij"]["f"]), wT(p["agg_ij"]["h"]),
        wT(p["gru_gr1"]["gate"]), wT(p["gru_gr1"]["r1"]), wT(p["gru_gr1"]["r2"]),
        wT(p["gru_gr2"]["gate"]), wT(p["gru_gr2"]["r1"]), wT(p["gru_gr2"]["r2"]),
    ])
    hw = jnp.zeros((D, 128), F32)
    hw = hw.at[:, 0:2].set(jnp.transpose(p["d"]["w"], (1, 0)))
    hw = hw.at[:, 2:4].set(jnp.transpose(p["w"]["w"], (1, 0)))
    hw = hw.astype(BF16)
    hb = jnp.zeros((D,), F32)
    hb = hb.at[0:2].set(p["d"]["b"])
    hb = hb.at[2:4].set(p["w"]["b"])
    bias = jnp.stack([
        p["corr0"]["b"], p["corr1"]["b"], p["corr_ln"]["w"], p["corr_ln"]["b"],
        p["corr2"]["b"], p["norm"]["w"], p["norm"]["b"],
        p["c1"]["l1"]["b"], p["c1"]["l2"]["b"],
        p["c2"]["l1"]["b"], p["c2"]["l2"]["b"],
        p["agg_kk"]["g"]["b"], p["agg_kk"]["f"]["b"], p["agg_kk"]["h"]["b"],
        p["agg_ij"]["g"]["b"], p["agg_ij"]["f"]["b"], p["agg_ij"]["h"]["b"],
        p["gru_ln1"]["w"], p["gru_ln1"]["b"],
        p["gru_gr1"]["gate"]["b"], p["gru_gr1"]["r1"]["b"], p["gru_gr1"]["r2"]["b"],
        p["gru_ln2"]["w"], p["gru_ln2"]["b"],
        p["gru_gr2"]["gate"]["b"], p["gru_gr2"]["r1"]["b"], p["gru_gr2"]["r2"]["b"],
        hb,
    ])

    def _in_blk(i):
        return (jnp.minimum(i, NB - 1), 0)

    def _in_blk3(i):
        return (jnp.minimum(i, NB - 1), 0, 0)

    def _out_blk(i):
        return (jnp.maximum(i - 6 * NB, 0), 0)

    def _c0(i):
        return (0, 0)

    def _c03(i):
        return (0, 0, 0)

    grid = (7 * NB,)
    net5, head = pl.pallas_call(
        _mega,
        grid=grid,
        in_specs=[
            pl.BlockSpec((R, CD), _in_blk),
            pl.BlockSpec((R, D), _in_blk),
            pl.BlockSpec((R, D), _in_blk),
            pl.BlockSpec((1, N), _c0),
            pl.BlockSpec((1, N), _c0),
            pl.BlockSpec((1, N), _c0),
            pl.BlockSpec((1, 1, R), _in_blk3),
            pl.BlockSpec((1, 1, R), _in_blk3),
            pl.BlockSpec((CD, D), _c0),
            pl.BlockSpec((18, D, D), _c03),
            pl.BlockSpec((D, 128), _c0),
            pl.BlockSpec((28, D), _c0),
        ],
        out_specs=[
            pl.BlockSpec((R, D), _out_blk),
            pl.BlockSpec((R, 128), _out_blk),
        ],
        out_shape=[_shape((N, D)), _shape((N, 128))],
        scratch_shapes=[
            pltpu.VMEM((N, D), F32), pltpu.VMEM((N, D), F32),
            pltpu.VMEM((N, D), F32), pltpu.VMEM((N, D), F32),
            pltpu.VMEM((N, D), F32),
            pltpu.VMEM((N, D), BF16), pltpu.VMEM((N, D), BF16),
            pltpu.VMEM((N, N), BF16),
            pltpu.VMEM((NB, R), jnp.int32), pltpu.VMEM((NB, R), jnp.int32),
        ],
    )(corr2d, net2d, inp2d, kkf, jjf, iif, kkb, jjb,
      jnp.transpose(p["corr0"]["w"], (1, 0)).astype(BF16), wstk, hw, bias)

    net_out = net5.reshape(1, N, D)
    d_out = head[:, 0:2].reshape(1, N, 2)
    w_out = head[:, 2:4].reshape(1, N, 2)
    return net_out, d_out, w_out
